# baseline (speedup 1.0000x reference)
import functools

import jax
import jax.numpy as jnp
from jax import lax
from jax.experimental import pallas as pl
from jax.experimental.pallas import tpu as pltpu
from jax.experimental.pallas import tpu_sc as plsc

EMBED = 64
NC = 2
NS = 16
NW = NC * NS

CHUNK = 512
KT = EMBED // 8
ITC = CHUNK // 128


@functools.lru_cache(maxsize=None)
def _build(batch, tlen):
    B = batch * tlen
    b_per_w = B // NW
    nchunks = b_per_w // CHUNK
    chunks_per_j = batch // CHUNK
    n_it = batch // 128
    assert b_per_w % CHUNK == 0 and batch % CHUNK == 0

    mesh = plsc.VectorSubcoreMesh(core_axis_name="c", subcore_axis_name="s")

    @functools.partial(
        pl.kernel,
        mesh=mesh,
        out_type=jax.ShapeDtypeStruct((tlen, KT, n_it, 8, 128), jnp.float32),
        compiler_params=pltpu.CompilerParams(
            use_tc_tiling_on_sc=False, needs_layout_passes=False
        ),
        scratch_types=[
            pltpu.VMEM((b_per_w,), jnp.int32),
            pltpu.VMEM((CHUNK, EMBED), jnp.float32),
            pltpu.VMEM((KT, ITC, 8, 128), jnp.float32),
            pltpu.SemaphoreType.DMA,
        ],
    )
    def k(table_hbm, idx_hbm, out_hbm, idx_v, rows, outt, sg):
        wid = lax.axis_index("s") * NC + lax.axis_index("c")
        base = wid * b_per_w
        pltpu.sync_copy(idx_hbm.at[pl.ds(base, b_per_w)], idx_v)
        lanes = lax.iota(jnp.int32, 16)

        def chunk_body(g, carry):
            c = wid * nchunks + g
            j = c // chunks_per_j
            it0 = (c % chunks_per_j) * ITC
            off = pl.multiple_of(g * CHUNK, 8)
            pltpu.async_copy(
                table_hbm.at[idx_v.at[pl.ds(off, CHUNK)]], rows, sg
            ).wait()

            def tr_body(q, carry2):
                itl = q // 8
                ii0 = (q % 8) * 16
                row_vec = q * 16 + lanes
                for kt in range(KT):
                    for kk in range(8):
                        col = kt * 8 + kk
                        val = plsc.load_gather(rows, [row_vec, lanes * 0 + col])
                        outt[kt, itl, kk, pl.ds(ii0, 16)] = val
                return carry2

            lax.fori_loop(0, CHUNK // 16, tr_body, 0)
            pltpu.sync_copy(outt, out_hbm.at[j, :, pl.ds(it0, ITC)])
            return carry

        lax.fori_loop(0, nchunks, chunk_body, 0)

    return k


@jax.jit
def kernel(x, table):
    b, t = x.shape
    flat = x.T.reshape(b * t)
    out5 = _build(b, t)(table, flat)
    return out5.transpose(2, 4, 0, 1, 3).reshape(b, t, EMBED)


# speedup vs baseline: 1.3041x; 1.3041x over previous
import functools

import jax
import jax.numpy as jnp
from jax import lax
from jax.experimental import pallas as pl
from jax.experimental.pallas import tpu as pltpu
from jax.experimental.pallas import tpu_sc as plsc

EMBED = 64
NC = 2
NS = 16
NW = NC * NS

CHUNK = 512
KT = EMBED // 8
ITC = CHUNK // 128


@functools.lru_cache(maxsize=None)
def _build(batch, tlen):
    B = batch * tlen
    b_per_w = B // NW
    nchunks = b_per_w // CHUNK
    chunks_per_j = batch // CHUNK
    n_it = batch // 128
    assert b_per_w % CHUNK == 0 and batch % CHUNK == 0

    mesh = plsc.VectorSubcoreMesh(core_axis_name="c", subcore_axis_name="s")

    @functools.partial(
        pl.kernel,
        mesh=mesh,
        out_type=jax.ShapeDtypeStruct((tlen, KT, n_it, 8, 128), jnp.float32),
        compiler_params=pltpu.CompilerParams(
            use_tc_tiling_on_sc=False, needs_layout_passes=False
        ),
        scratch_types=[
            pltpu.VMEM((b_per_w,), jnp.int32),
            pltpu.VMEM((CHUNK, EMBED), jnp.float32),
            pltpu.VMEM((KT, ITC, 8, 128), jnp.float32),
            pltpu.SemaphoreType.DMA,
        ],
    )
    def k(table_hbm, idx_hbm, out_hbm, idx_v, rows, outt, sg):
        wid = lax.axis_index("s") * NC + lax.axis_index("c")
        base = wid * b_per_w
        pltpu.sync_copy(idx_hbm.at[pl.ds(base, b_per_w)], idx_v)
        lanes = lax.iota(jnp.int32, 16)

        def chunk_body(g, carry):
            c = wid * nchunks + g
            j = c // chunks_per_j
            it0 = (c % chunks_per_j) * ITC
            off = pl.multiple_of(g * CHUNK, 8)
            pltpu.async_copy(
                table_hbm.at[idx_v.at[pl.ds(off, CHUNK)]], rows, sg
            ).wait()

            @plsc.parallel_loop(0, CHUNK // 16, unroll=2)
            def tr_body(q):
                itl = q // 8
                ii0 = (q % 8) * 16
                row_vec = q * 16 + lanes
                for kt in range(KT):
                    for kk in range(8):
                        col = kt * 8 + kk
                        val = plsc.load_gather(rows, [row_vec, lanes * 0 + col])
                        outt[kt, itl, kk, pl.ds(ii0, 16)] = val
            pltpu.sync_copy(outt, out_hbm.at[j, :, pl.ds(it0, ITC)])
            return carry

        lax.fori_loop(0, nchunks, chunk_body, 0)

    return k


@jax.jit
def kernel(x, table):
    b, t = x.shape
    flat = x.T.reshape(b * t)
    out5 = _build(b, t)(table, flat)
    return out5.transpose(2, 4, 0, 1, 3).reshape(b, t, EMBED)


# speedup vs baseline: 2.0821x; 1.5965x over previous
import functools

import jax
import jax.numpy as jnp
from jax import lax
from jax.experimental import pallas as pl
from jax.experimental.pallas import tpu as pltpu
from jax.experimental.pallas import tpu_sc as plsc

EMBED = 64
NC = 2
NS = 16
NW = NC * NS

CHUNK = 512


@functools.lru_cache(maxsize=None)
def _build(batch, tlen):
    B = batch * tlen
    b_per_w = B // NW
    nchunks = b_per_w // CHUNK
    chunks_per_j = batch // CHUNK
    assert b_per_w % CHUNK == 0 and batch % CHUNK == 0

    mesh = plsc.VectorSubcoreMesh(core_axis_name="c", subcore_axis_name="s")

    @functools.partial(
        pl.kernel,
        mesh=mesh,
        out_type=jax.ShapeDtypeStruct((batch, tlen, 128), jnp.float32),
        compiler_params=pltpu.CompilerParams(
            use_tc_tiling_on_sc=False, needs_layout_passes=False
        ),
        scratch_types=[
            pltpu.VMEM((b_per_w,), jnp.int32),
            pltpu.VMEM((CHUNK, EMBED), jnp.float32),
            pltpu.SemaphoreType.DMA,
        ],
    )
    def k(table_hbm, idx_hbm, out_hbm, idx_v, rows, sg):
        wid = lax.axis_index("s") * NC + lax.axis_index("c")
        base = wid * b_per_w
        pltpu.sync_copy(idx_hbm.at[pl.ds(base, b_per_w)], idx_v)

        def body(g, carry):
            c = wid * nchunks + g
            j = c // chunks_per_j
            i0 = (c % chunks_per_j) * CHUNK
            off = pl.multiple_of(g * CHUNK, 8)
            pltpu.async_copy(
                table_hbm.at[idx_v.at[pl.ds(off, CHUNK)]], rows, sg
            ).wait()
            pltpu.sync_copy(
                rows, out_hbm.at[pl.ds(i0, CHUNK), j, pl.ds(0, EMBED)]
            )
            return carry

        lax.fori_loop(0, nchunks, body, 0)

    return k


@jax.jit
def kernel(x, table):
    b, t = x.shape
    flat = x.T.reshape(b * t)
    outp = _build(b, t)(table, flat)
    return outp[:, :, :EMBED]


# speedup vs baseline: 2.1408x; 1.0282x over previous
import functools

import jax
import jax.numpy as jnp
from jax import lax
from jax.experimental import pallas as pl
from jax.experimental.pallas import tpu as pltpu
from jax.experimental.pallas import tpu_sc as plsc

EMBED = 64
NC = 2
NS = 16
NW = NC * NS

CHUNK = 512


@functools.lru_cache(maxsize=None)
def _build(batch, tlen):
    B = batch * tlen
    b_per_w = B // NW
    nchunks = b_per_w // CHUNK
    chunks_per_j = batch // CHUNK
    assert b_per_w % CHUNK == 0 and batch % CHUNK == 0

    mesh = plsc.VectorSubcoreMesh(core_axis_name="c", subcore_axis_name="s")

    @functools.partial(
        pl.kernel,
        mesh=mesh,
        out_type=jax.ShapeDtypeStruct((batch, tlen, 128), jnp.float32),
        compiler_params=pltpu.CompilerParams(use_tc_tiling_on_sc=False),
        scratch_types=[
            pltpu.VMEM((b_per_w,), jnp.int32),
            pltpu.VMEM((CHUNK, EMBED), jnp.float32),
            pltpu.VMEM((CHUNK, EMBED), jnp.float32),
            pltpu.SemaphoreType.DMA,
            pltpu.SemaphoreType.DMA,
            pltpu.SemaphoreType.DMA,
            pltpu.SemaphoreType.DMA,
        ],
    )
    def k(table_hbm, idx_hbm, out_hbm, idx_v, r0, r1, sg0, sg1, so0, so1):
        wid = lax.axis_index("s") * NC + lax.axis_index("c")
        base = wid * b_per_w
        pltpu.sync_copy(idx_hbm.at[pl.ds(base, b_per_w)], idx_v)
        rows = (r0, r1)
        gsem = (sg0, sg1)
        osem = (so0, so1)

        def gather(g, buf):
            off = pl.multiple_of(g * CHUNK, 8)
            return pltpu.async_copy(
                table_hbm.at[idx_v.at[pl.ds(off, CHUNK)]], rows[buf],
                gsem[buf],
            )

        def out_copy(g, buf):
            c = wid * nchunks + g
            j = c // chunks_per_j
            i0 = (c % chunks_per_j) * CHUNK
            return pltpu.async_copy(
                rows[buf],
                out_hbm.at[pl.ds(i0, CHUNK), j, pl.ds(0, EMBED)],
                osem[buf],
            )

        def drain_out(g, buf):
            c = wid * nchunks + g
            j = c // chunks_per_j
            i0 = (c % chunks_per_j) * CHUNK
            pltpu.make_async_copy(
                rows[buf],
                out_hbm.at[pl.ds(i0, CHUNK), j, pl.ds(0, EMBED)],
                osem[buf],
            ).wait()

        def body(i, carry):
            for b in (0, 1):
                g = 2 * i + b

                @pl.when(i > 0)
                def _():
                    drain_out(g, b)

                gather(g, b).wait()
                out_copy(g, b)
            return carry

        lax.fori_loop(0, nchunks // 2, body, 0)
        for b in (0, 1):
            drain_out(nchunks - 2 + b, b)

    return k


@jax.jit
def kernel(x, table):
    b, t = x.shape
    flat = x.T.reshape(b * t)
    outp = _build(b, t)(table, flat)
    return outp[:, :, :EMBED]
